# baseline (device time: 386606 ns/iter reference)
import jax
import jax.numpy as jnp
from jax import lax
from jax.experimental import pallas as pl
from jax.experimental.pallas import tpu as pltpu

N_DEV = 4
M_CHUNK = 1024
SLAB_N = 2048
HALF_N = SLAB_N // 2
N_SLABS = 8192 // SLAB_N


def kernel(x, w_mat):
    m, k_per = x.shape
    _, n = w_mat.shape
    x = x.astype(jnp.bfloat16)
    w_mat = w_mat.astype(jnp.bfloat16)

    def body(x_ref, w_ref, out_ref,
             send_r, send_l, recv_r, recv_l, sems_r, sems_l,
             recv_sems_r, recv_sems_l):
        s = pl.program_id(0)
        my = lax.axis_index("i")
        left = (my - 1) % N_DEV
        right = (my + 1) % N_DEV

        @pl.when(s == 0)
        def _():
            barrier_sem = pltpu.get_barrier_semaphore()
            for nbr in [left, right]:
                pl.semaphore_signal(
                    barrier_sem, inc=1,
                    device_id=(nbr,), device_id_type=pl.DeviceIdType.MESH,
                )
            pl.semaphore_wait(barrier_sem, 2)

        def partial_r(c):
            xc = x_ref[pl.ds(c * M_CHUNK, M_CHUNK), :]
            return jnp.dot(xc, w_ref[:, :HALF_N],
                           preferred_element_type=jnp.float32)

        def partial_l(c):
            xc = x_ref[pl.ds(c * M_CHUNK, M_CHUNK), :]
            return jnp.dot(xc, w_ref[:, HALF_N:],
                           preferred_element_type=jnp.float32)

        send_r[:, :] = partial_r((my - 1) % N_DEV).astype(jnp.bfloat16)
        send_l[:, :] = partial_l((my + 1) % N_DEV).astype(jnp.bfloat16)

        for h in range(N_DEV - 1):
            rdma_r = pltpu.make_async_remote_copy(
                src_ref=send_r,
                dst_ref=recv_r.at[h],
                send_sem=sems_r.at[h],
                recv_sem=recv_sems_r.at[h],
                device_id=(right,),
                device_id_type=pl.DeviceIdType.MESH,
            )
            rdma_l = pltpu.make_async_remote_copy(
                src_ref=send_l,
                dst_ref=recv_l.at[h],
                send_sem=sems_l.at[h],
                recv_sem=recv_sems_l.at[h],
                device_id=(left,),
                device_id_type=pl.DeviceIdType.MESH,
            )
            rdma_r.start()
            rdma_l.start()
            p_r = partial_r((my - 2 - h) % N_DEV)
            p_l = partial_l((my + 2 + h) % N_DEV)
            rdma_r.wait()
            rdma_l.wait()
            if h < N_DEV - 2:
                send_r[:, :] = (recv_r[h] + p_r).astype(jnp.bfloat16)
                send_l[:, :] = (recv_l[h] + p_l).astype(jnp.bfloat16)
            else:
                out_ref[:, :HALF_N] = jnp.maximum(recv_r[h] + p_r, 0.0)
                out_ref[:, HALF_N:] = jnp.maximum(recv_l[h] + p_l, 0.0)

    grid = (N_SLABS,)
    return pl.pallas_call(
        body,
        grid=grid,
        out_shape=jax.ShapeDtypeStruct((M_CHUNK, n), jnp.float32),
        in_specs=[
            pl.BlockSpec((m, k_per), lambda s: (0, 0)),
            pl.BlockSpec((k_per, SLAB_N), lambda s: (0, s)),
        ],
        out_specs=pl.BlockSpec((M_CHUNK, SLAB_N), lambda s: (0, s)),
        scratch_shapes=[
            pltpu.VMEM((M_CHUNK, HALF_N), jnp.bfloat16),
            pltpu.VMEM((M_CHUNK, HALF_N), jnp.bfloat16),
            pltpu.VMEM((N_DEV - 1, M_CHUNK, HALF_N), jnp.bfloat16),
            pltpu.VMEM((N_DEV - 1, M_CHUNK, HALF_N), jnp.bfloat16),
            pltpu.SemaphoreType.DMA((N_DEV - 1,)),
            pltpu.SemaphoreType.DMA((N_DEV - 1,)),
            pltpu.SemaphoreType.DMA((N_DEV - 1,)),
            pltpu.SemaphoreType.DMA((N_DEV - 1,)),
        ],
        compiler_params=pltpu.CompilerParams(
            collective_id=0,
            dimension_semantics=("arbitrary",),
            vmem_limit_bytes=64 * 1024 * 1024,
        ),
    )(x, w_mat)


# device time: 351248 ns/iter; 1.1007x vs baseline; 1.1007x over previous
import jax
import jax.numpy as jnp
from jax import lax
from jax.experimental import pallas as pl
from jax.experimental.pallas import tpu as pltpu

N_DEV = 4
N_HOP = N_DEV - 1
M_CHUNK = 1024
SLAB_N = 2048
HALF_N = SLAB_N // 2
SUB = 2
SUB_N = HALF_N // SUB
N_SLABS = 8192 // SLAB_N
ORDER = (0, 2, 1, 3)


def kernel(x, w_mat):
    m, k_per = x.shape
    _, n = w_mat.shape
    x = x.astype(jnp.bfloat16)
    w_mat = w_mat.astype(jnp.bfloat16)

    def body(x_ref, w_ref, out_ref, send_buf, recv_buf, send_sems, recv_sems):
        s = pl.program_id(0)
        my = lax.axis_index("i")
        left = (my - 1) % N_DEV
        right = (my + 1) % N_DEV

        @pl.when(s == 0)
        def _():
            barrier_sem = pltpu.get_barrier_semaphore()
            for nbr in [left, right]:
                pl.semaphore_signal(
                    barrier_sem, inc=1,
                    device_id=(nbr,), device_id_type=pl.DeviceIdType.MESH,
                )
            pl.semaphore_wait(barrier_sem, 2)

        def info(si):
            cw = si < SUB
            col = si * SUB_N if cw else HALF_N + (si - SUB) * SUB_N
            return cw, col, (right if cw else left)

        def partial(si, h):
            cw, col, _ = info(si)
            c = (my - 2 - h) % N_DEV if cw else (my + 2 + h) % N_DEV
            xc = x_ref[pl.ds(c * M_CHUNK, M_CHUNK), :]
            return jnp.dot(xc, w_ref[:, col:col + SUB_N],
                           preferred_element_type=jnp.float32)

        def make(si, h):
            _, _, dst = info(si)
            return pltpu.make_async_remote_copy(
                src_ref=send_buf.at[si],
                dst_ref=recv_buf.at[si, h],
                send_sem=send_sems.at[si, h],
                recv_sem=recv_sems.at[si, h],
                device_id=(dst,),
                device_id_type=pl.DeviceIdType.MESH,
            )

        rdmas = {}
        for si in ORDER:
            send_buf[si, :, :] = partial(si, -1).astype(jnp.bfloat16)
            rdmas[(si, 0)] = make(si, 0)
            rdmas[(si, 0)].start()

        for h in range(N_HOP):
            for si in ORDER:
                _, col, _ = info(si)
                p = partial(si, h)
                rdmas[(si, h)].wait()
                if h < N_HOP - 1:
                    send_buf[si, :, :] = (recv_buf[si, h] + p).astype(
                        jnp.bfloat16)
                    rdmas[(si, h + 1)] = make(si, h + 1)
                    rdmas[(si, h + 1)].start()
                else:
                    out_ref[:, col:col + SUB_N] = jnp.maximum(
                        recv_buf[si, h] + p, 0.0)

    grid = (N_SLABS,)
    n_str = 2 * SUB
    return pl.pallas_call(
        body,
        grid=grid,
        out_shape=jax.ShapeDtypeStruct((M_CHUNK, n), jnp.float32),
        in_specs=[
            pl.BlockSpec((m, k_per), lambda s: (0, 0)),
            pl.BlockSpec((k_per, SLAB_N), lambda s: (0, s)),
        ],
        out_specs=pl.BlockSpec((M_CHUNK, SLAB_N), lambda s: (0, s)),
        scratch_shapes=[
            pltpu.VMEM((n_str, M_CHUNK, SUB_N), jnp.bfloat16),
            pltpu.VMEM((n_str, N_HOP, M_CHUNK, SUB_N), jnp.bfloat16),
            pltpu.SemaphoreType.DMA((n_str, N_HOP)),
            pltpu.SemaphoreType.DMA((n_str, N_HOP)),
        ],
        compiler_params=pltpu.CompilerParams(
            collective_id=0,
            dimension_semantics=("arbitrary",),
            vmem_limit_bytes=64 * 1024 * 1024,
        ),
    )(x, w_mat)


# device time: 348738 ns/iter; 1.1086x vs baseline; 1.0072x over previous
import jax
import jax.numpy as jnp
from jax import lax
from jax.experimental import pallas as pl
from jax.experimental.pallas import tpu as pltpu

N_DEV = 4
N_HOP = N_DEV - 1
M_CHUNK = 1024
HALF_TOT = 4096
SUB_N = 512
N_STRIPS = HALF_TOT // SUB_N
N_PER_CHAIN = N_STRIPS // 2
CHAINS = ((True, 0), (False, 0), (True, 1), (False, 1))


def kernel(x, w_mat):
    m, k_per = x.shape
    _, n = w_mat.shape
    x = x.astype(jnp.bfloat16)
    w_mat = w_mat.astype(jnp.bfloat16)

    def body(x_ref, w_hbm, out_hbm, w_buf, send_buf, recv_buf, out_stage,
             w_sems, send_sems, recv_sems, out_sems):
        my = lax.axis_index("i")
        left = (my - 1) % N_DEV
        right = (my + 1) % N_DEV

        def col0(ci, j):
            cw, parity = CHAINS[ci]
            k = 2 * j + parity
            return (0 if cw else HALF_TOT) + k * SUB_N

        wcopies = {}
        outcopies = {}
        rdmas = {}

        def issue_wcopy(ci, j):
            c = pltpu.make_async_copy(
                w_hbm.at[:, pl.ds(col0(ci, j), SUB_N)],
                w_buf.at[ci, j % 2],
                w_sems.at[ci, j % 2],
            )
            c.start()
            wcopies[(ci, j)] = c

        for ci in range(len(CHAINS)):
            issue_wcopy(ci, 0)
            issue_wcopy(ci, 1)

        barrier_sem = pltpu.get_barrier_semaphore()
        for nbr in [left, right]:
            pl.semaphore_signal(
                barrier_sem, inc=1,
                device_id=(nbr,), device_id_type=pl.DeviceIdType.MESH,
            )
        pl.semaphore_wait(barrier_sem, 2)

        def partial(ci, j, h):
            cw, _ = CHAINS[ci]
            c = (my - 2 - h) % N_DEV if cw else (my + 2 + h) % N_DEV
            xc = x_ref[pl.ds(c * M_CHUNK, M_CHUNK), :]
            return jnp.dot(xc, w_buf[ci, j % 2],
                           preferred_element_type=jnp.float32)

        def make_rdma(ci, h):
            cw, _ = CHAINS[ci]
            return pltpu.make_async_remote_copy(
                src_ref=send_buf.at[ci],
                dst_ref=recv_buf.at[ci, h],
                send_sem=send_sems.at[ci, h],
                recv_sem=recv_sems.at[ci, h],
                device_id=(right if cw else left,),
                device_id_type=pl.DeviceIdType.MESH,
            )

        def ev_init(ci, j):
            wcopies[(ci, j)].wait()
            send_buf[ci, :, :] = partial(ci, j, -1).astype(jnp.bfloat16)
            r = make_rdma(ci, 0)
            r.start()
            rdmas[(ci, j, 0)] = r

        def ev_hop(ci, j, h):
            p = partial(ci, j, h)
            rdmas[(ci, j, h)].wait()
            if h < N_HOP - 1:
                send_buf[ci, :, :] = (recv_buf[ci, h] + p).astype(
                    jnp.bfloat16)
                r = make_rdma(ci, h + 1)
                r.start()
                rdmas[(ci, j, h + 1)] = r
            else:
                if ci in outcopies:
                    outcopies[ci].wait()
                out_stage[ci, :, :] = jnp.maximum(recv_buf[ci, h] + p, 0.0)
                oc = pltpu.make_async_copy(
                    out_stage.at[ci],
                    out_hbm.at[:, pl.ds(col0(ci, j), SUB_N)],
                    out_sems.at[ci],
                )
                oc.start()
                outcopies[ci] = oc
                if j + 2 < N_PER_CHAIN:
                    issue_wcopy(ci, j + 2)

        for t in range(N_PER_CHAIN * (N_HOP + 1)):
            j, phase = divmod(t, N_HOP + 1)
            for ci in range(len(CHAINS)):
                if phase == 0:
                    ev_init(ci, j)
                else:
                    ev_hop(ci, j, phase - 1)

        for ci in range(len(CHAINS)):
            outcopies[ci].wait()

    n_ch = len(CHAINS)
    return pl.pallas_call(
        body,
        out_shape=jax.ShapeDtypeStruct((M_CHUNK, n), jnp.float32),
        in_specs=[
            pl.BlockSpec(memory_space=pltpu.MemorySpace.VMEM),
            pl.BlockSpec(memory_space=pl.ANY),
        ],
        out_specs=pl.BlockSpec(memory_space=pl.ANY),
        scratch_shapes=[
            pltpu.VMEM((n_ch, 2, k_per, SUB_N), jnp.bfloat16),
            pltpu.VMEM((n_ch, M_CHUNK, SUB_N), jnp.bfloat16),
            pltpu.VMEM((n_ch, N_HOP, M_CHUNK, SUB_N), jnp.bfloat16),
            pltpu.VMEM((n_ch, M_CHUNK, SUB_N), jnp.float32),
            pltpu.SemaphoreType.DMA((n_ch, 2)),
            pltpu.SemaphoreType.DMA((n_ch, N_HOP)),
            pltpu.SemaphoreType.DMA((n_ch, N_HOP)),
            pltpu.SemaphoreType.DMA((n_ch,)),
        ],
        compiler_params=pltpu.CompilerParams(
            collective_id=0,
            vmem_limit_bytes=60 * 1024 * 1024,
        ),
    )(x, w_mat)


# device time: 324253 ns/iter; 1.1923x vs baseline; 1.0755x over previous
import jax
import jax.numpy as jnp
from jax import lax
from jax.experimental import pallas as pl
from jax.experimental.pallas import tpu as pltpu

N_DEV = 4
N_HOP = N_DEV - 1
M_CHUNK = 1024
HALF_TOT = 4096
SUB_N = 512
N_STRIPS = HALF_TOT // SUB_N
N_PER_CHAIN = N_STRIPS // 2
CHAINS = ((True, 0), (False, 0), (True, 1), (False, 1))


def kernel(x, w_mat):
    m, k_per = x.shape
    _, n = w_mat.shape

    def body(x_ref, w_hbm, out_hbm, w_buf, send_buf, recv_buf, out_stage,
             w_sems, send_sems, recv_sems, out_sems):
        my = lax.axis_index("i")
        left = (my - 1) % N_DEV
        right = (my + 1) % N_DEV

        def col0(ci, j):
            cw, parity = CHAINS[ci]
            k = 2 * j + parity
            return (0 if cw else HALF_TOT) + k * SUB_N

        wcopies = {}
        outcopies = {}
        rdmas = {}

        def issue_wcopy(ci, j):
            c = pltpu.make_async_copy(
                w_hbm.at[:, pl.ds(col0(ci, j), SUB_N)],
                w_buf.at[ci, j % 2],
                w_sems.at[ci, j % 2],
            )
            c.start()
            wcopies[(ci, j)] = c

        for ci in range(len(CHAINS)):
            issue_wcopy(ci, 0)
            issue_wcopy(ci, 1)

        barrier_sem = pltpu.get_barrier_semaphore()
        for nbr in [left, right]:
            pl.semaphore_signal(
                barrier_sem, inc=1,
                device_id=(nbr,), device_id_type=pl.DeviceIdType.MESH,
            )
        pl.semaphore_wait(barrier_sem, 2)

        def partial(ci, j, h):
            cw, _ = CHAINS[ci]
            c = (my - 2 - h) % N_DEV if cw else (my + 2 + h) % N_DEV
            xc = x_ref[pl.ds(c * M_CHUNK, M_CHUNK), :]
            return jnp.dot(xc, w_buf[ci, j % 2],
                           preferred_element_type=jnp.float32)

        def make_rdma(ci, h):
            cw, _ = CHAINS[ci]
            return pltpu.make_async_remote_copy(
                src_ref=send_buf.at[ci],
                dst_ref=recv_buf.at[ci, h],
                send_sem=send_sems.at[ci, h],
                recv_sem=recv_sems.at[ci, h],
                device_id=(right if cw else left,),
                device_id_type=pl.DeviceIdType.MESH,
            )

        def ev_init(ci, j):
            wcopies[(ci, j)].wait()
            send_buf[ci, :, :] = partial(ci, j, -1).astype(jnp.bfloat16)
            r = make_rdma(ci, 0)
            r.start()
            rdmas[(ci, j, 0)] = r

        def ev_hop(ci, j, h):
            p = partial(ci, j, h)
            rdmas[(ci, j, h)].wait()
            if h < N_HOP - 1:
                send_buf[ci, :, :] = (recv_buf[ci, h] + p).astype(
                    jnp.bfloat16)
                r = make_rdma(ci, h + 1)
                r.start()
                rdmas[(ci, j, h + 1)] = r
            else:
                if ci in outcopies:
                    outcopies[ci].wait()
                out_stage[ci, :, :] = jnp.maximum(recv_buf[ci, h] + p, 0.0)
                oc = pltpu.make_async_copy(
                    out_stage.at[ci],
                    out_hbm.at[:, pl.ds(col0(ci, j), SUB_N)],
                    out_sems.at[ci],
                )
                oc.start()
                outcopies[ci] = oc
                if j + 2 < N_PER_CHAIN:
                    issue_wcopy(ci, j + 2)

        for t in range(N_PER_CHAIN * (N_HOP + 1)):
            j, phase = divmod(t, N_HOP + 1)
            for ci in range(len(CHAINS)):
                if phase == 0:
                    ev_init(ci, j)
                else:
                    ev_hop(ci, j, phase - 1)

        for ci in range(len(CHAINS)):
            outcopies[ci].wait()

    n_ch = len(CHAINS)
    return pl.pallas_call(
        body,
        out_shape=jax.ShapeDtypeStruct((M_CHUNK, n), jnp.float32),
        in_specs=[
            pl.BlockSpec(memory_space=pltpu.MemorySpace.VMEM),
            pl.BlockSpec(memory_space=pl.ANY),
        ],
        out_specs=pl.BlockSpec(memory_space=pl.ANY),
        scratch_shapes=[
            pltpu.VMEM((n_ch, 2, k_per, SUB_N), jnp.float32),
            pltpu.VMEM((n_ch, M_CHUNK, SUB_N), jnp.bfloat16),
            pltpu.VMEM((n_ch, N_HOP, M_CHUNK, SUB_N), jnp.bfloat16),
            pltpu.VMEM((n_ch, M_CHUNK, SUB_N), jnp.float32),
            pltpu.SemaphoreType.DMA((n_ch, 2)),
            pltpu.SemaphoreType.DMA((n_ch, N_HOP)),
            pltpu.SemaphoreType.DMA((n_ch, N_HOP)),
            pltpu.SemaphoreType.DMA((n_ch,)),
        ],
        compiler_params=pltpu.CompilerParams(
            collective_id=0,
            vmem_limit_bytes=67_000_000,
        ),
    )(x, w_mat)
